# Initial kernel scaffold; baseline (speedup 1.0000x reference)
import jax
import jax.numpy as jnp
from jax.experimental import pallas as pl


def kernel(input, kernel_bank, buckets):
    raise NotImplementedError("write your pallas kernel here")


# speedup vs baseline: 15.8943x; 15.8943x over previous
import functools

import jax
import jax.numpy as jnp
from jax import lax
from jax.experimental import pallas as pl
from jax.experimental.pallas import tpu as pltpu
from jax.experimental.pallas import tpu_sc as plsc

B, C, H, W = 2, 96, 384, 384
E = 64
K = 3
T = K * K
N = H * W


_NC, _NS = 2, 16
_NW = _NC * _NS
_PPW = (B * N) // _NW
_WPB = _NW // B

def _sc_wm_body(bank_hbm, bk_hbm, wm_hbm, bank_v, idx_v, wm_v):
    wid = lax.axis_index("s") * _NC + lax.axis_index("c")
    b = wid // _WPB
    off = (wid % _WPB) * _PPW

    pltpu.sync_copy(bank_hbm.at[pl.ds(b * T * E, T * E)], bank_v)
    pltpu.sync_copy(bk_hbm.at[pl.ds(b * N + off, _PPW)], idx_v)

    def step(i, carry):
        idx = idx_v[pl.ds(i * 16, 16)]
        for t in range(T):
            vals = plsc.load_gather(bank_v, [idx + (t * E)])
            wm_v[pl.ds(t * _PPW + i * 16, 16)] = vals
        return carry

    lax.fori_loop(0, _PPW // 16, step, 0)

    for t in range(T):
        pltpu.sync_copy(wm_v.at[pl.ds(t * _PPW, _PPW)],
                        wm_hbm.at[pl.ds((b * T + t) * N + off, _PPW)])


@functools.cache
def _sc_wm():
    mesh = plsc.VectorSubcoreMesh(core_axis_name="c", subcore_axis_name="s",
                                  num_cores=_NC)
    return pl.kernel(
        _sc_wm_body,
        mesh=mesh,
        out_type=jax.ShapeDtypeStruct((B * T * N,), jnp.float32),
        scratch_types=[
            pltpu.VMEM((T * E,), jnp.float32),
            pltpu.VMEM((_PPW,), jnp.int32),
            pltpu.VMEM((T * _PPW,), jnp.float32),
        ],
        compiler_params=pltpu.CompilerParams(needs_layout_passes=False),
    )


_CB = 4
_NCB = C // _CB
_RS = 48
_NS_TC = H // _RS


def _row_strip(x, r0, i):
    g0 = r0 + i - 1
    zrow = jnp.zeros((_CB, 1, W), jnp.float32)
    if g0 < 0:
        return jnp.concatenate([zrow, x[:, 0:_RS - 1]], axis=1)
    if g0 + _RS > H:
        return jnp.concatenate([x[:, g0:H], zrow], axis=1)
    return x[:, g0:g0 + _RS]


def _col_shift(s, j):
    zcol = jnp.zeros((_CB, _RS, 1), jnp.float32)
    if j == 0:
        return jnp.concatenate([zcol, s[:, :, :W - 1]], axis=2)
    if j == 2:
        return jnp.concatenate([s[:, :, 1:], zcol], axis=2)
    return s


def _conv_body(wm_ref, x_ref, o_ref):
    x = x_ref[0]
    for si in range(_NS_TC):
        r0 = si * _RS
        acc = None
        for i in range(K):
            s = _row_strip(x, r0, i)
            for j in range(K):
                term = _col_shift(s, j) * wm_ref[0, i * K + j, r0:r0 + _RS][None]
                acc = term if acc is None else acc + term
        o_ref[0, :, r0:r0 + _RS] = acc


def _conv(wm, x):
    return pl.pallas_call(
        _conv_body,
        grid=(B, _NCB),
        in_specs=[
            pl.BlockSpec((1, T, H, W), lambda b, c: (b, 0, 0, 0)),
            pl.BlockSpec((1, _CB, H, W), lambda b, c: (b, c, 0, 0)),
        ],
        out_specs=pl.BlockSpec((1, _CB, H, W), lambda b, c: (b, c, 0, 0)),
        out_shape=jax.ShapeDtypeStruct((B, C, H, W), jnp.float32),
    )(wm, x)


def kernel(input, kernel_bank, buckets):
    bank_t = jnp.transpose(kernel_bank.reshape(B, E, T), (0, 2, 1)).reshape(B * T * E)
    wm = _sc_wm()(bank_t, buckets.reshape(B * N))
    return _conv(wm.reshape(B, T, H, W), input)


# speedup vs baseline: 16.1388x; 1.0154x over previous
import functools

import jax
import jax.numpy as jnp
from jax import lax
from jax.experimental import pallas as pl
from jax.experimental.pallas import tpu as pltpu
from jax.experimental.pallas import tpu_sc as plsc

B, C, H, W = 2, 96, 384, 384
E = 64
K = 3
T = K * K
N = H * W


_NC, _NS = 2, 16
_NW = _NC * _NS
_PPW = (B * N) // _NW
_WPB = _NW // B

def _sc_wm_body(bank_hbm, bk_hbm, wm_hbm, bank_v, idx_v, wm_v):
    wid = lax.axis_index("s") * _NC + lax.axis_index("c")
    b = wid // _WPB
    off = (wid % _WPB) * _PPW

    pltpu.sync_copy(bank_hbm.at[pl.ds(b * T * E, T * E)], bank_v)
    pltpu.sync_copy(bk_hbm.at[pl.ds(b * N + off, _PPW)], idx_v)

    def step(i, carry):
        idx = idx_v[pl.ds(i * 16, 16)]
        for t in range(T):
            vals = plsc.load_gather(bank_v, [idx + (t * E)])
            wm_v[pl.ds(t * _PPW + i * 16, 16)] = vals
        return carry

    lax.fori_loop(0, _PPW // 16, step, 0)

    for t in range(T):
        pltpu.sync_copy(wm_v.at[pl.ds(t * _PPW, _PPW)],
                        wm_hbm.at[pl.ds((b * T + t) * N + off, _PPW)])


@functools.cache
def _sc_wm():
    mesh = plsc.VectorSubcoreMesh(core_axis_name="c", subcore_axis_name="s",
                                  num_cores=_NC)
    return pl.kernel(
        _sc_wm_body,
        mesh=mesh,
        out_type=jax.ShapeDtypeStruct((B * T * N,), jnp.float32),
        scratch_types=[
            pltpu.VMEM((T * E,), jnp.float32),
            pltpu.VMEM((_PPW,), jnp.int32),
            pltpu.VMEM((T * _PPW,), jnp.float32),
        ],
        compiler_params=pltpu.CompilerParams(needs_layout_passes=False),
    )


_CB = 4
_NCB = C // _CB
_RS = 48
_NS_TC = H // _RS


def _row_strip(x, r0, i):
    g0 = r0 + i - 1
    zrow = jnp.zeros((_CB, 1, W), jnp.float32)
    if g0 < 0:
        return jnp.concatenate([zrow, x[:, 0:_RS - 1]], axis=1)
    if g0 + _RS > H:
        return jnp.concatenate([x[:, g0:H], zrow], axis=1)
    return x[:, g0:g0 + _RS]


def _conv_body(wm_ref, x_ref, o_ref, s0a_ref, s2a_ref, s0b_ref, s2b_ref):
    x = x_ref[0]
    zc1 = jnp.zeros((_RS, 1), jnp.float32)
    zcol = jnp.zeros((_CB, _RS, 1), jnp.float32)
    bufs = ((s0a_ref, s2a_ref), (s0b_ref, s2b_ref))
    bufs[0][0][...] = _row_strip(x, 0, 0)
    bufs[0][1][...] = _row_strip(x, 0, 2)
    for si in range(_NS_TC):
        r0 = si * _RS
        cur, nxt = bufs[si % 2], bufs[(si + 1) % 2]
        if si + 1 < _NS_TC:
            nxt[0][...] = _row_strip(x, r0 + _RS, 0)
            nxt[1][...] = _row_strip(x, r0 + _RS, 2)
        xs = (cur[0][...], x[:, r0:r0 + _RS], cur[1][...])
        acc = None
        for j in range(K):
            cj = None
            for i in range(K):
                wmv = wm_ref[0, i * K + j, r0:r0 + _RS, :]
                if j == 0:
                    wmv = jnp.concatenate([wmv[:, 1:], zc1], axis=1)
                elif j == 2:
                    wmv = jnp.concatenate([zc1, wmv[:, :W - 1]], axis=1)
                term = xs[i] * wmv[None]
                cj = term if cj is None else cj + term
            if j == 0:
                cj = jnp.concatenate([zcol, cj[:, :, :W - 1]], axis=2)
            elif j == 2:
                cj = jnp.concatenate([cj[:, :, 1:], zcol], axis=2)
            acc = cj if acc is None else acc + cj
        o_ref[0, :, r0:r0 + _RS] = acc


def _conv(wm, x):
    return pl.pallas_call(
        _conv_body,
        grid=(B, _NCB),
        in_specs=[
            pl.BlockSpec((1, T, H, W), lambda b, c: (b, 0, 0, 0)),
            pl.BlockSpec((1, _CB, H, W), lambda b, c: (b, c, 0, 0)),
        ],
        out_specs=pl.BlockSpec((1, _CB, H, W), lambda b, c: (b, c, 0, 0)),
        out_shape=jax.ShapeDtypeStruct((B, C, H, W), jnp.float32),
        scratch_shapes=[pltpu.VMEM((_CB, _RS, W), jnp.float32)] * 4,
    )(wm, x)


def kernel(input, kernel_bank, buckets):
    bank_t = jnp.transpose(kernel_bank.reshape(B, E, T), (0, 2, 1)).reshape(B * T * E)
    wm = _sc_wm()(bank_t, buckets.reshape(B * N))
    return _conv(wm.reshape(B, T, H, W), input)


# speedup vs baseline: 19.8109x; 1.2275x over previous
import functools

import jax
import jax.numpy as jnp
from jax import lax
from jax.experimental import pallas as pl
from jax.experimental.pallas import tpu as pltpu
from jax.experimental.pallas import tpu_sc as plsc

B, C, H, W = 2, 96, 384, 384
E = 64
K = 3
T = K * K
N = H * W


_NC, _NS = 2, 16
_NW = _NC * _NS
_PPW = (B * N) // _NW
_WPB = _NW // B

def _sc_wm_body(bank_hbm, bk_hbm, wm_hbm, bank_v, idx_v, wm_v):
    wid = lax.axis_index("s") * _NC + lax.axis_index("c")
    b = wid // _WPB
    off = (wid % _WPB) * _PPW

    pltpu.sync_copy(bank_hbm.at[pl.ds(b * T * E, T * E)], bank_v)
    pltpu.sync_copy(bk_hbm.at[pl.ds(b * N + off, _PPW)], idx_v)

    def step(i, carry):
        idx = idx_v[pl.ds(i * 16, 16)]
        for t in range(T):
            vals = plsc.load_gather(bank_v, [idx + (t * E)])
            wm_v[pl.ds(t * _PPW + i * 16, 16)] = vals
        return carry

    lax.fori_loop(0, _PPW // 16, step, 0)

    for t in range(T):
        pltpu.sync_copy(wm_v.at[pl.ds(t * _PPW, _PPW)],
                        wm_hbm.at[pl.ds((b * T + t) * N + off, _PPW)])


@functools.cache
def _sc_wm():
    mesh = plsc.VectorSubcoreMesh(core_axis_name="c", subcore_axis_name="s",
                                  num_cores=_NC)
    return pl.kernel(
        _sc_wm_body,
        mesh=mesh,
        out_type=jax.ShapeDtypeStruct((B * T * N,), jnp.float32),
        scratch_types=[
            pltpu.VMEM((T * E,), jnp.float32),
            pltpu.VMEM((_PPW,), jnp.int32),
            pltpu.VMEM((T * _PPW,), jnp.float32),
        ],
        compiler_params=pltpu.CompilerParams(needs_layout_passes=False),
    )


_CB = 8
_NCB = C // _CB
_RS = 96
_NS_TC = H // _RS


def _row_strip(x, r0, i):
    g0 = r0 + i - 1
    zrow = jnp.zeros((_CB, 1, W), jnp.float32)
    if g0 < 0:
        return jnp.concatenate([zrow, x[:, 0:_RS - 1]], axis=1)
    if g0 + _RS > H:
        return jnp.concatenate([x[:, g0:H], zrow], axis=1)
    return x[:, g0:g0 + _RS]


def _conv_body(wm_ref, x_ref, o_ref, s0a_ref, s2a_ref, s0b_ref, s2b_ref):
    x = x_ref[0]
    zc1 = jnp.zeros((_RS, 1), jnp.float32)
    zcol = jnp.zeros((_CB, _RS, 1), jnp.float32)
    bufs = ((s0a_ref, s2a_ref), (s0b_ref, s2b_ref))
    bufs[0][0][...] = _row_strip(x, 0, 0)
    bufs[0][1][...] = _row_strip(x, 0, 2)
    for si in range(_NS_TC):
        r0 = si * _RS
        cur, nxt = bufs[si % 2], bufs[(si + 1) % 2]
        if si + 1 < _NS_TC:
            nxt[0][...] = _row_strip(x, r0 + _RS, 0)
            nxt[1][...] = _row_strip(x, r0 + _RS, 2)
        xs = (cur[0][...], x[:, r0:r0 + _RS], cur[1][...])
        acc = None
        for j in range(K):
            cj = None
            for i in range(K):
                wmv = wm_ref[0, i * K + j, r0:r0 + _RS, :]
                if j == 0:
                    wmv = jnp.concatenate([wmv[:, 1:], zc1], axis=1)
                elif j == 2:
                    wmv = jnp.concatenate([zc1, wmv[:, :W - 1]], axis=1)
                term = xs[i] * wmv[None]
                cj = term if cj is None else cj + term
            if j == 0:
                cj = jnp.concatenate([zcol, cj[:, :, :W - 1]], axis=2)
            elif j == 2:
                cj = jnp.concatenate([cj[:, :, 1:], zcol], axis=2)
            acc = cj if acc is None else acc + cj
        o_ref[0, :, r0:r0 + _RS] = acc


def _conv(wm, x):
    return pl.pallas_call(
        _conv_body,
        grid=(B, _NCB),
        in_specs=[
            pl.BlockSpec((1, T, H, W), lambda b, c: (b, 0, 0, 0)),
            pl.BlockSpec((1, _CB, H, W), lambda b, c: (b, c, 0, 0)),
        ],
        out_specs=pl.BlockSpec((1, _CB, H, W), lambda b, c: (b, c, 0, 0)),
        out_shape=jax.ShapeDtypeStruct((B, C, H, W), jnp.float32),
        scratch_shapes=[pltpu.VMEM((_CB, _RS, W), jnp.float32)] * 4,
    )(wm, x)


def kernel(input, kernel_bank, buckets):
    bank_t = jnp.transpose(kernel_bank.reshape(B, E, T), (0, 2, 1)).reshape(B * T * E)
    wm = _sc_wm()(bank_t, buckets.reshape(B * N))
    return _conv(wm.reshape(B, T, H, W), input)


# speedup vs baseline: 22.1802x; 1.1196x over previous
import functools

import jax
import jax.numpy as jnp
from jax import lax
from jax.experimental import pallas as pl
from jax.experimental.pallas import tpu as pltpu
from jax.experimental.pallas import tpu_sc as plsc

B, C, H, W = 2, 96, 384, 384
E = 64
K = 3
T = K * K
N = H * W


_NC, _NS = 2, 16
_NW = _NC * _NS
_PPW = (B * N) // _NW
_WPB = _NW // B

def _sc_wm_body(bank_hbm, bk_hbm, wm_hbm, bank_v, idx_v, wm_v):
    wid = lax.axis_index("s") * _NC + lax.axis_index("c")
    b = wid // _WPB
    off = (wid % _WPB) * _PPW

    pltpu.sync_copy(bank_hbm.at[pl.ds(b * T * E, T * E)], bank_v)
    pltpu.sync_copy(bk_hbm.at[pl.ds(b * N + off, _PPW)], idx_v)

    @plsc.parallel_loop(0, _PPW // 16, unroll=8)
    def _gather_step(i):
        idx = idx_v[pl.ds(i * 16, 16)]
        for t in range(T):
            vals = plsc.load_gather(bank_v, [idx + (t * E)])
            wm_v[pl.ds(t * _PPW + i * 16, 16)] = vals

    for t in range(T):
        pltpu.sync_copy(wm_v.at[pl.ds(t * _PPW, _PPW)],
                        wm_hbm.at[pl.ds((b * T + t) * N + off, _PPW)])


@functools.cache
def _sc_wm():
    mesh = plsc.VectorSubcoreMesh(core_axis_name="c", subcore_axis_name="s",
                                  num_cores=_NC)
    return pl.kernel(
        _sc_wm_body,
        mesh=mesh,
        out_type=jax.ShapeDtypeStruct((B * T * N,), jnp.float32),
        scratch_types=[
            pltpu.VMEM((T * E,), jnp.float32),
            pltpu.VMEM((_PPW,), jnp.int32),
            pltpu.VMEM((T * _PPW,), jnp.float32),
        ],
        compiler_params=pltpu.CompilerParams(needs_layout_passes=False),
    )


_CB = 8
_NCB = C // _CB
_RS = 96
_NS_TC = H // _RS


def _row_strip(x, r0, i):
    g0 = r0 + i - 1
    zrow = jnp.zeros((_CB, 1, W), jnp.float32)
    if g0 < 0:
        return jnp.concatenate([zrow, x[:, 0:_RS - 1]], axis=1)
    if g0 + _RS > H:
        return jnp.concatenate([x[:, g0:H], zrow], axis=1)
    return x[:, g0:g0 + _RS]


def _conv_body(wm_ref, x_ref, o_ref, s0a_ref, s2a_ref, s0b_ref, s2b_ref):
    x = x_ref[0]
    zc1 = jnp.zeros((_RS, 1), jnp.float32)
    zcol = jnp.zeros((_CB, _RS, 1), jnp.float32)
    bufs = ((s0a_ref, s2a_ref), (s0b_ref, s2b_ref))
    bufs[0][0][...] = _row_strip(x, 0, 0)
    bufs[0][1][...] = _row_strip(x, 0, 2)
    for si in range(_NS_TC):
        r0 = si * _RS
        cur, nxt = bufs[si % 2], bufs[(si + 1) % 2]
        if si + 1 < _NS_TC:
            nxt[0][...] = _row_strip(x, r0 + _RS, 0)
            nxt[1][...] = _row_strip(x, r0 + _RS, 2)
        xs = (cur[0][...], x[:, r0:r0 + _RS], cur[1][...])
        acc = None
        for j in range(K):
            cj = None
            for i in range(K):
                wmv = wm_ref[0, i * K + j, r0:r0 + _RS, :]
                if j == 0:
                    wmv = jnp.concatenate([wmv[:, 1:], zc1], axis=1)
                elif j == 2:
                    wmv = jnp.concatenate([zc1, wmv[:, :W - 1]], axis=1)
                term = xs[i] * wmv[None]
                cj = term if cj is None else cj + term
            if j == 0:
                cj = jnp.concatenate([zcol, cj[:, :, :W - 1]], axis=2)
            elif j == 2:
                cj = jnp.concatenate([cj[:, :, 1:], zcol], axis=2)
            acc = cj if acc is None else acc + cj
        o_ref[0, :, r0:r0 + _RS] = acc


def _conv(wm, x):
    return pl.pallas_call(
        _conv_body,
        grid=(B, _NCB),
        in_specs=[
            pl.BlockSpec((1, T, H, W), lambda b, c: (b, 0, 0, 0)),
            pl.BlockSpec((1, _CB, H, W), lambda b, c: (b, c, 0, 0)),
        ],
        out_specs=pl.BlockSpec((1, _CB, H, W), lambda b, c: (b, c, 0, 0)),
        out_shape=jax.ShapeDtypeStruct((B, C, H, W), jnp.float32),
        scratch_shapes=[pltpu.VMEM((_CB, _RS, W), jnp.float32)] * 4,
    )(wm, x)


def kernel(input, kernel_bank, buckets):
    bank_t = jnp.transpose(kernel_bank.reshape(B, E, T), (0, 2, 1)).reshape(B * T * E)
    wm = _sc_wm()(bank_t, buckets.reshape(B * N))
    return _conv(wm.reshape(B, T, H, W), input)


# speedup vs baseline: 23.6441x; 1.0660x over previous
import functools

import jax
import jax.numpy as jnp
from jax import lax
from jax.experimental import pallas as pl
from jax.experimental.pallas import tpu as pltpu
from jax.experimental.pallas import tpu_sc as plsc

B, C, H, W = 2, 96, 384, 384
E = 64
K = 3
T = K * K
N = H * W


_NC, _NS = 2, 16
_NW = _NC * _NS
_PPW = (B * N) // _NW
_WPB = _NW // B
_RPW = _PPW // W

def _sc_wm_body(bank_hbm, bk_hbm, wm_hbm, bank_v, idx_v, wm_v):
    wid = lax.axis_index("s") * _NC + lax.axis_index("c")
    b = wid // _WPB
    off = (wid % _WPB) * _PPW

    pltpu.sync_copy(bank_hbm.at[pl.ds(b * T * E, T * E)], bank_v)
    pltpu.sync_copy(bk_hbm.at[pl.ds(b * N + off, _PPW)], idx_v)

    @plsc.parallel_loop(0, _PPW // 16, unroll=8)
    def _gather_step(i):
        r = i // (W // 16)
        c16 = (i % (W // 16)) * 16
        idx = idx_v[pl.ds(i * 16, 16)]
        for t in range(T):
            vals = plsc.load_gather(bank_v, [idx + (t * E)])
            wm_v[t, r, pl.ds(c16, 16)] = vals

    row0 = (wid % _WPB) * _RPW
    for t in range(T):
        pltpu.sync_copy(wm_v.at[t], wm_hbm.at[b, t, pl.ds(row0, _RPW)])


@functools.cache
def _sc_wm():
    mesh = plsc.VectorSubcoreMesh(core_axis_name="c", subcore_axis_name="s",
                                  num_cores=_NC)
    return pl.kernel(
        _sc_wm_body,
        mesh=mesh,
        out_type=jax.ShapeDtypeStruct((B, T, H, W), jnp.float32),
        scratch_types=[
            pltpu.VMEM((T * E,), jnp.float32),
            pltpu.VMEM((_PPW,), jnp.int32),
            pltpu.VMEM((T, _RPW, W), jnp.float32),
        ],
        compiler_params=pltpu.CompilerParams(needs_layout_passes=False),
    )


_CB = 8
_NCB = C // _CB
_RS = 96
_NS_TC = H // _RS


def _row_strip(x, r0, i):
    g0 = r0 + i - 1
    zrow = jnp.zeros((_CB, 1, W), jnp.float32)
    if g0 < 0:
        return jnp.concatenate([zrow, x[:, 0:_RS - 1]], axis=1)
    if g0 + _RS > H:
        return jnp.concatenate([x[:, g0:H], zrow], axis=1)
    return x[:, g0:g0 + _RS]


def _conv_body(wm_ref, x_ref, o_ref, s0a_ref, s2a_ref, s0b_ref, s2b_ref):
    x = x_ref[0]
    zc1 = jnp.zeros((_RS, 1), jnp.float32)
    zcol = jnp.zeros((_CB, _RS, 1), jnp.float32)
    bufs = ((s0a_ref, s2a_ref), (s0b_ref, s2b_ref))
    bufs[0][0][...] = _row_strip(x, 0, 0)
    bufs[0][1][...] = _row_strip(x, 0, 2)
    for si in range(_NS_TC):
        r0 = si * _RS
        cur, nxt = bufs[si % 2], bufs[(si + 1) % 2]
        if si + 1 < _NS_TC:
            nxt[0][...] = _row_strip(x, r0 + _RS, 0)
            nxt[1][...] = _row_strip(x, r0 + _RS, 2)
        xs = (cur[0][...], x[:, r0:r0 + _RS], cur[1][...])
        acc = None
        for j in range(K):
            cj = None
            for i in range(K):
                wmv = wm_ref[0, i * K + j, r0:r0 + _RS, :]
                if j == 0:
                    wmv = jnp.concatenate([wmv[:, 1:], zc1], axis=1)
                elif j == 2:
                    wmv = jnp.concatenate([zc1, wmv[:, :W - 1]], axis=1)
                term = xs[i] * wmv[None]
                cj = term if cj is None else cj + term
            if j == 0:
                cj = jnp.concatenate([zcol, cj[:, :, :W - 1]], axis=2)
            elif j == 2:
                cj = jnp.concatenate([cj[:, :, 1:], zcol], axis=2)
            acc = cj if acc is None else acc + cj
        o_ref[0, :, r0:r0 + _RS] = acc


def _conv(wm, x):
    return pl.pallas_call(
        _conv_body,
        grid=(B, _NCB),
        in_specs=[
            pl.BlockSpec((1, T, H, W), lambda b, c: (b, 0, 0, 0)),
            pl.BlockSpec((1, _CB, H, W), lambda b, c: (b, c, 0, 0)),
        ],
        out_specs=pl.BlockSpec((1, _CB, H, W), lambda b, c: (b, c, 0, 0)),
        out_shape=jax.ShapeDtypeStruct((B, C, H, W), jnp.float32),
        scratch_shapes=[pltpu.VMEM((_CB, _RS, W), jnp.float32)] * 4,
    )(wm, x)


def kernel(input, kernel_bank, buckets):
    bank_t = jnp.transpose(kernel_bank.reshape(B, E, T), (0, 2, 1)).reshape(B * T * E)
    wm = _sc_wm()(bank_t, buckets.reshape(B * N))
    return _conv(wm, input)


# speedup vs baseline: 26.0139x; 1.1002x over previous
import functools

import jax
import jax.numpy as jnp
from jax import lax
from jax.experimental import pallas as pl
from jax.experimental.pallas import tpu as pltpu
from jax.experimental.pallas import tpu_sc as plsc

B, C, H, W = 2, 96, 384, 384
E = 64
K = 3
T = K * K
N = H * W


_NC, _NS = 2, 16
_NW = _NC * _NS
_PPW = (B * N) // _NW
_WPB = _NW // B
_RPW = _PPW // W

def _sc_wm_body(bank_hbm, bk_hbm, wm_hbm, bank_v, idx_v, wm_v):
    wid = lax.axis_index("s") * _NC + lax.axis_index("c")
    b = wid // _WPB
    off = (wid % _WPB) * _PPW

    pltpu.sync_copy(bank_hbm.at[pl.ds(b * T * E, T * E)], bank_v)
    pltpu.sync_copy(bk_hbm.at[pl.ds(b * N + off, _PPW)], idx_v)

    iota16 = lax.iota(jnp.int32, 16)

    @plsc.parallel_loop(0, _PPW // 32, unroll=4)
    def _gather_step(i):
        r = i // (W // 32)
        c32 = (i % (W // 32)) * 32
        base = i * 32
        idx_e = plsc.load_gather(idx_v, [base + 2 * iota16])
        idx_o = plsc.load_gather(idx_v, [base + 2 * iota16 + 1])
        for t in range(T):
            ve = plsc.load_gather(bank_v, [idx_e + (t * E)])
            vo = plsc.load_gather(bank_v, [idx_o + (t * E)])
            wm_v[t, r, pl.ds(c32, 32)] = plsc.pack(
                ve, vo, format=plsc.PackFormat.INTERLEAVED)

    row0 = (wid % _WPB) * _RPW
    for t in range(T):
        pltpu.sync_copy(wm_v.at[t], wm_hbm.at[b, t, pl.ds(row0, _RPW)])


@functools.cache
def _sc_wm():
    mesh = plsc.VectorSubcoreMesh(core_axis_name="c", subcore_axis_name="s",
                                  num_cores=_NC)
    return pl.kernel(
        _sc_wm_body,
        mesh=mesh,
        out_type=jax.ShapeDtypeStruct((B, T, H, W), jnp.bfloat16),
        scratch_types=[
            pltpu.VMEM((T * E,), jnp.float32),
            pltpu.VMEM((_PPW,), jnp.int32),
            pltpu.VMEM((T, _RPW, W), jnp.bfloat16),
        ],
        compiler_params=pltpu.CompilerParams(needs_layout_passes=False),
    )


_CB = 12
_NCB = C // _CB
_RS = 96
_NS_TC = H // _RS


def _row_strip(x, r0, i):
    g0 = r0 + i - 1
    zrow = jnp.zeros((_CB, 1, W), jnp.float32)
    if g0 < 0:
        return jnp.concatenate([zrow, x[:, 0:_RS - 1]], axis=1)
    if g0 + _RS > H:
        return jnp.concatenate([x[:, g0:H], zrow], axis=1)
    return x[:, g0:g0 + _RS]


def _conv_body(wm_ref, x_ref, o_ref, s0a_ref, s2a_ref, s0b_ref, s2b_ref):
    x = x_ref[0]
    zc1 = jnp.zeros((_RS, 1), jnp.float32)
    zcol = jnp.zeros((_CB, _RS, 1), jnp.float32)
    bufs = ((s0a_ref, s2a_ref), (s0b_ref, s2b_ref))
    bufs[0][0][...] = _row_strip(x, 0, 0)
    bufs[0][1][...] = _row_strip(x, 0, 2)
    for si in range(_NS_TC):
        r0 = si * _RS
        cur, nxt = bufs[si % 2], bufs[(si + 1) % 2]
        if si + 1 < _NS_TC:
            nxt[0][...] = _row_strip(x, r0 + _RS, 0)
            nxt[1][...] = _row_strip(x, r0 + _RS, 2)
        xs = (cur[0][...], x[:, r0:r0 + _RS], cur[1][...])
        acc = None
        for j in range(K):
            cj = None
            for i in range(K):
                wmv = wm_ref[0, i * K + j, r0:r0 + _RS, :].astype(jnp.float32)
                if j == 0:
                    wmv = jnp.concatenate([wmv[:, 1:], zc1], axis=1)
                elif j == 2:
                    wmv = jnp.concatenate([zc1, wmv[:, :W - 1]], axis=1)
                term = xs[i] * wmv[None]
                cj = term if cj is None else cj + term
            if j == 0:
                cj = jnp.concatenate([zcol, cj[:, :, :W - 1]], axis=2)
            elif j == 2:
                cj = jnp.concatenate([cj[:, :, 1:], zcol], axis=2)
            acc = cj if acc is None else acc + cj
        o_ref[0, :, r0:r0 + _RS] = acc


def _conv(wm, x):
    return pl.pallas_call(
        _conv_body,
        grid=(B, _NCB),
        in_specs=[
            pl.BlockSpec((1, T, H, W), lambda b, c: (b, 0, 0, 0)),
            pl.BlockSpec((1, _CB, H, W), lambda b, c: (b, c, 0, 0)),
        ],
        out_specs=pl.BlockSpec((1, _CB, H, W), lambda b, c: (b, c, 0, 0)),
        out_shape=jax.ShapeDtypeStruct((B, C, H, W), jnp.float32),
        scratch_shapes=[pltpu.VMEM((_CB, _RS, W), jnp.float32)] * 4,
    )(wm, x)


def kernel(input, kernel_bank, buckets):
    bank_t = jnp.transpose(kernel_bank.reshape(B, E, T), (0, 2, 1)).reshape(B * T * E)
    wm = _sc_wm()(bank_t, buckets.reshape(B * N))
    return _conv(wm, input)
